# Initial kernel scaffold; baseline (speedup 1.0000x reference)
import jax
import jax.numpy as jnp
from jax.experimental import pallas as pl


def kernel(x, y):
    raise NotImplementedError("write your pallas kernel here")


# speedup vs baseline: 90.0993x; 90.0993x over previous
import functools

import jax
import jax.numpy as jnp
from jax import lax
from jax.experimental import pallas as pl
from jax.experimental.pallas import tpu as pltpu
from jax.experimental.pallas import tpu_sc as plsc

_NT = 4096
_NS = 768
_ROWS = 256
_WORKERS = 32
_PER_W = _NS // _WORKERS


def _tc_prep_body(xt_ref, yt_ref, syn_ref, obs_ref):
    xv = xt_ref[...]
    yv = yt_ref[...]
    mind = jnp.minimum(
        jnp.min(xv, axis=1, keepdims=True),
        jnp.min(yv, axis=1, keepdims=True),
    )
    lane = lax.broadcasted_iota(jnp.int32, (_ROWS, _NT), 1)
    valid = lane < (_NT - 1)
    zcol = jnp.zeros((_ROWS, 1), jnp.float32)
    for v, out in ((xv, syn_ref), (yv, obs_ref)):
        s = v - mind
        s_next = jnp.concatenate([s[:, 1:], zcol], axis=1)
        tz = jnp.where(valid, (s + s_next) * 0.5, 0.0)
        c = tz
        d = 1
        while d < _NT:
            shifted = jnp.concatenate(
                [jnp.zeros((_ROWS, d), jnp.float32), c[:, : _NT - d]], axis=1
            )
            c = c + shifted
            d *= 2
        total = jnp.sum(c, axis=1, keepdims=True) - c[:, _NT - 1 : _NT]
        out[...] = jnp.where(valid, c / total, 0.0)


def _tc_prep(xt, yt):
    grid = _NS // _ROWS
    spec = pl.BlockSpec((_ROWS, _NT), lambda i: (i, 0))
    return pl.pallas_call(
        _tc_prep_body,
        grid=(grid,),
        in_specs=[spec, spec],
        out_specs=[spec, spec],
        out_shape=[
            jax.ShapeDtypeStruct((_NS, _NT), jnp.float32),
            jax.ShapeDtypeStruct((_NS, _NT), jnp.float32),
        ],
    )(xt, yt)


def _sc_search_body(syn_hbm, obs_hbm, out_hbm, syn_v, obs_v, acc_v):
    info = plsc.get_sparse_core_info()
    nc = info.num_cores
    wid = lax.axis_index("s") * nc + lax.axis_index("c")
    lane = lax.iota(jnp.int32, 16)

    def col_body(j, acc):
        c = wid * _PER_W + j
        pltpu.sync_copy(syn_hbm.at[c], syn_v)
        pltpu.sync_copy(obs_hbm.at[c], obs_v)

        def chunk_body(k, a):
            q = syn_v[pl.ds(k * 16, 16)]
            lo = jnp.zeros((16,), jnp.int32)
            hi = jnp.full((16,), _NT - 1, jnp.int32)
            for _ in range(12):
                mid = (lo + hi) >> 1
                v = plsc.load_gather(obs_v, [mid])
                cond = v < q
                lo = jnp.where(cond, mid + 1, lo)
                hi = jnp.where(cond, hi, mid)
            diff = (k * 16 + 1 + lane - lo).astype(jnp.float32)
            return a + diff * diff * q

        return lax.fori_loop(0, _NT // 16, chunk_body, acc)

    acc = lax.fori_loop(0, _PER_W, col_body, jnp.zeros((16,), jnp.float32))
    acc_v[...] = acc
    pltpu.sync_copy(acc_v, out_hbm.at[wid])


def _sc_search(syn, obs):
    mesh = plsc.VectorSubcoreMesh(core_axis_name="c", subcore_axis_name="s")
    kern = functools.partial(
        pl.kernel,
        out_type=jax.ShapeDtypeStruct((_WORKERS, 16), jnp.float32),
        mesh=mesh,
        scratch_types=[
            pltpu.VMEM((_NT,), jnp.float32),
            pltpu.VMEM((_NT,), jnp.float32),
            pltpu.VMEM((16,), jnp.float32),
        ],
        compiler_params=pltpu.CompilerParams(needs_layout_passes=False),
    )(_sc_search_body)
    return kern(syn, obs)


def kernel(x, y):
    xt = x.reshape(_NT, -1).T
    yt = y.reshape(_NT, -1).T
    syn, obs = _tc_prep(xt, yt)
    part = _sc_search(syn, obs)
    return jnp.sum(part)


# speedup vs baseline: 100.4778x; 1.1152x over previous
import functools

import jax
import jax.numpy as jnp
from jax import lax
from jax.experimental import pallas as pl
from jax.experimental.pallas import tpu as pltpu
from jax.experimental.pallas import tpu_sc as plsc

_NT = 4096
_NS = 768
_ROWS = 256
_WORKERS = 32
_PER_W = _NS // _WORKERS


def _tc_prep_body(xt_ref, yt_ref, syn_ref, obs_ref):
    xv = xt_ref[...]
    yv = yt_ref[...]
    mind = jnp.minimum(
        jnp.min(xv, axis=1, keepdims=True),
        jnp.min(yv, axis=1, keepdims=True),
    )
    lane = lax.broadcasted_iota(jnp.int32, (_ROWS, _NT), 1)
    valid = lane < (_NT - 1)
    zcol = jnp.zeros((_ROWS, 1), jnp.float32)
    for v, out in ((xv, syn_ref), (yv, obs_ref)):
        s = v - mind
        s_next = jnp.concatenate([s[:, 1:], zcol], axis=1)
        tz = jnp.where(valid, (s + s_next) * 0.5, 0.0)
        c = tz
        d = 1
        while d < _NT:
            shifted = jnp.concatenate(
                [jnp.zeros((_ROWS, d), jnp.float32), c[:, : _NT - d]], axis=1
            )
            c = c + shifted
            d *= 2
        total = jnp.sum(c, axis=1, keepdims=True) - c[:, _NT - 1 : _NT]
        out[...] = jnp.where(valid, c / total, 0.0)


def _tc_prep(xt, yt):
    grid = _NS // _ROWS
    spec = pl.BlockSpec((_ROWS, _NT), lambda i: (i, 0))
    return pl.pallas_call(
        _tc_prep_body,
        grid=(grid,),
        in_specs=[spec, spec],
        out_specs=[spec, spec],
        out_shape=[
            jax.ShapeDtypeStruct((_NS, _NT), jnp.float32),
            jax.ShapeDtypeStruct((_NS, _NT), jnp.float32),
        ],
    )(xt, yt)


_UNROLL = 4


def _search_col(syn_v, obs_v, acc):
    lane = lax.iota(jnp.int32, 16)

    def chunk_body(k, a):
        for u in range(_UNROLL):
            ck = k * _UNROLL + u
            q = syn_v[pl.ds(ck * 16, 16)]
            lo = jnp.zeros((16,), jnp.int32)
            hi = jnp.full((16,), _NT - 1, jnp.int32)
            for _ in range(12):
                mid = (lo + hi) >> 1
                v = plsc.load_gather(obs_v, [mid])
                cond = v < q
                lo = jnp.where(cond, mid + 1, lo)
                hi = jnp.where(cond, hi, mid)
            diff = (ck * 16 + 1 + lane - lo).astype(jnp.float32)
            a = a + diff * diff * q
        return a

    return lax.fori_loop(0, _NT // (16 * _UNROLL), chunk_body, acc)


def _sc_search_body(
    syn_hbm, obs_hbm, out_hbm, syn0, obs0, syn1, obs1, acc_v, sem0, sem1
):
    info = plsc.get_sparse_core_info()
    nc = info.num_cores
    wid = lax.axis_index("s") * nc + lax.axis_index("c")
    base = wid * _PER_W
    last = base + _PER_W - 1

    pltpu.sync_copy(syn_hbm.at[base], syn0)
    pltpu.sync_copy(obs_hbm.at[base], obs0)

    def col2_body(jj, acc):
        c0 = base + jj * 2
        nxt = jnp.minimum(c0 + 1, last)
        h1 = pltpu.async_copy(syn_hbm.at[nxt], syn1, sem1)
        h2 = pltpu.async_copy(obs_hbm.at[nxt], obs1, sem1)
        acc = _search_col(syn0, obs0, acc)
        h1.wait()
        h2.wait()
        nxt2 = jnp.minimum(c0 + 2, last)
        h3 = pltpu.async_copy(syn_hbm.at[nxt2], syn0, sem0)
        h4 = pltpu.async_copy(obs_hbm.at[nxt2], obs0, sem0)
        acc = _search_col(syn1, obs1, acc)
        h3.wait()
        h4.wait()
        return acc

    acc = lax.fori_loop(
        0, _PER_W // 2, col2_body, jnp.zeros((16,), jnp.float32)
    )
    acc_v[...] = acc
    pltpu.sync_copy(acc_v, out_hbm.at[wid])


def _sc_search(syn, obs):
    mesh = plsc.VectorSubcoreMesh(core_axis_name="c", subcore_axis_name="s")
    kern = functools.partial(
        pl.kernel,
        out_type=jax.ShapeDtypeStruct((_WORKERS, 16), jnp.float32),
        mesh=mesh,
        scratch_types=[
            pltpu.VMEM((_NT,), jnp.float32),
            pltpu.VMEM((_NT,), jnp.float32),
            pltpu.VMEM((_NT,), jnp.float32),
            pltpu.VMEM((_NT,), jnp.float32),
            pltpu.VMEM((16,), jnp.float32),
            pltpu.SemaphoreType.DMA,
            pltpu.SemaphoreType.DMA,
        ],
        compiler_params=pltpu.CompilerParams(needs_layout_passes=False),
    )(_sc_search_body)
    return kern(syn, obs)


def kernel(x, y):
    xt = x.reshape(_NT, -1).T
    yt = y.reshape(_NT, -1).T
    syn, obs = _tc_prep(xt, yt)
    part = _sc_search(syn, obs)
    return jnp.sum(part)


# speedup vs baseline: 135.3230x; 1.3468x over previous
import functools

import jax
import jax.numpy as jnp
from jax import lax
from jax.experimental import pallas as pl
from jax.experimental.pallas import tpu as pltpu
from jax.experimental.pallas import tpu_sc as plsc

_NT = 4096
_NS = 768
_ROWS = 256
_WORKERS = 32
_PER_W = _NS // _WORKERS


def _tc_prep_body(xt_ref, yt_ref, syn_ref, obs_ref):
    xv = xt_ref[...]
    yv = yt_ref[...]
    mind = jnp.minimum(
        jnp.min(xv, axis=1, keepdims=True),
        jnp.min(yv, axis=1, keepdims=True),
    )
    lane = lax.broadcasted_iota(jnp.int32, (_ROWS, _NT), 1)
    valid = lane < (_NT - 1)
    zcol = jnp.zeros((_ROWS, 1), jnp.float32)
    ch = 128
    ia = lax.broadcasted_iota(jnp.int32, (ch, ch), 0)
    ib = lax.broadcasted_iota(jnp.int32, (ch, ch), 1)
    tri = (ia <= ib).astype(jnp.float32)
    for v, out in ((xv, syn_ref), (yv, obs_ref)):
        s = v - mind
        s_next = jnp.concatenate([s[:, 1:], zcol], axis=1)
        tz = jnp.where(valid, (s + s_next) * 0.5, 0.0)
        carry = jnp.zeros((_ROWS, 1), jnp.float32)
        pieces = []
        for t in range(_NT // ch):
            blk = tz[:, t * ch : (t + 1) * ch]
            cs = (
                jnp.dot(blk, tri, preferred_element_type=jnp.float32) + carry
            )
            carry = carry + jnp.sum(blk, axis=1, keepdims=True)
            pieces.append(cs)
        c = jnp.concatenate(pieces, axis=1)
        total = jnp.sum(c, axis=1, keepdims=True) - carry
        out[...] = jnp.where(valid, c / total, 0.0)


def _tc_prep(xt, yt):
    grid = _NS // _ROWS
    spec = pl.BlockSpec((_ROWS, _NT), lambda i: (i, 0))
    return pl.pallas_call(
        _tc_prep_body,
        grid=(grid,),
        in_specs=[spec, spec],
        out_specs=[spec, spec],
        out_shape=[
            jax.ShapeDtypeStruct((_NS, _NT), jnp.float32),
            jax.ShapeDtypeStruct((_NS, _NT), jnp.float32),
        ],
    )(xt, yt)


_UNROLL = 4


def _search_col(syn_v, obs_v, acc):
    lane = lax.iota(jnp.int32, 16)

    def chunk_body(k, a):
        for u in range(_UNROLL):
            ck = k * _UNROLL + u
            q = syn_v[pl.ds(ck * 16, 16)]
            pos = jnp.zeros((16,), jnp.int32)
            d = _NT // 2
            while d >= 1:
                probe = pos + (d - 1)
                v = plsc.load_gather(obs_v, [probe])
                pos = jnp.where(v < q, probe + 1, pos)
                d //= 2
            diff = (ck * 16 + 1 + lane - pos).astype(jnp.float32)
            a = a + diff * diff * q
        return a

    return lax.fori_loop(0, _NT // (16 * _UNROLL), chunk_body, acc)


def _sc_search_body(
    syn_hbm, obs_hbm, out_hbm, syn0, obs0, syn1, obs1, acc_v, sem0, sem1
):
    info = plsc.get_sparse_core_info()
    nc = info.num_cores
    wid = lax.axis_index("s") * nc + lax.axis_index("c")
    base = wid * _PER_W
    last = base + _PER_W - 1

    pltpu.sync_copy(syn_hbm.at[base], syn0)
    pltpu.sync_copy(obs_hbm.at[base], obs0)

    def col2_body(jj, acc):
        c0 = base + jj * 2
        nxt = jnp.minimum(c0 + 1, last)
        h1 = pltpu.async_copy(syn_hbm.at[nxt], syn1, sem1)
        h2 = pltpu.async_copy(obs_hbm.at[nxt], obs1, sem1)
        acc = _search_col(syn0, obs0, acc)
        h1.wait()
        h2.wait()
        nxt2 = jnp.minimum(c0 + 2, last)
        h3 = pltpu.async_copy(syn_hbm.at[nxt2], syn0, sem0)
        h4 = pltpu.async_copy(obs_hbm.at[nxt2], obs0, sem0)
        acc = _search_col(syn1, obs1, acc)
        h3.wait()
        h4.wait()
        return acc

    acc = lax.fori_loop(
        0, _PER_W // 2, col2_body, jnp.zeros((16,), jnp.float32)
    )
    acc_v[...] = acc
    pltpu.sync_copy(acc_v, out_hbm.at[wid])


def _sc_search(syn, obs):
    mesh = plsc.VectorSubcoreMesh(core_axis_name="c", subcore_axis_name="s")
    kern = functools.partial(
        pl.kernel,
        out_type=jax.ShapeDtypeStruct((_WORKERS, 16), jnp.float32),
        mesh=mesh,
        scratch_types=[
            pltpu.VMEM((_NT,), jnp.float32),
            pltpu.VMEM((_NT,), jnp.float32),
            pltpu.VMEM((_NT,), jnp.float32),
            pltpu.VMEM((_NT,), jnp.float32),
            pltpu.VMEM((16,), jnp.float32),
            pltpu.SemaphoreType.DMA,
            pltpu.SemaphoreType.DMA,
        ],
        compiler_params=pltpu.CompilerParams(needs_layout_passes=False),
    )(_sc_search_body)
    return kern(syn, obs)


def kernel(x, y):
    xt = x.reshape(_NT, -1).T
    yt = y.reshape(_NT, -1).T
    syn, obs = _tc_prep(xt, yt)
    part = _sc_search(syn, obs)
    return jnp.sum(part)
